# Initial kernel scaffold; baseline (speedup 1.0000x reference)
import jax
import jax.numpy as jnp
from jax.experimental import pallas as pl


def kernel(inputs, sym_table, pos_table):
    raise NotImplementedError("write your pallas kernel here")


# speedup vs baseline: 2.3609x; 2.3609x over previous
import functools

import jax
import jax.numpy as jnp
from jax import lax
from jax.experimental import pallas as pl
from jax.experimental.pallas import tpu as pltpu
from jax.experimental.pallas import tpu_sc as plsc

NC = 2
NS = 16
NW = NC * NS
LANES = 16


def _bias_body(sym_ref, pos_ref, out_ref):
    colsum = jnp.sum(sym_ref[...], axis=0, keepdims=True)
    out_ref[...] = pos_ref[...] - colsum * (1.0 / sym_ref.shape[0])


@functools.partial(jax.jit, static_argnames=("B", "S", "D"))
def _sc_embed(idx_flat, sym_table, bias, *, B, S, D):
    ROWS = B // NW
    C0 = 104
    C1 = S - C0
    mesh = plsc.VectorSubcoreMesh(
        core_axis_name="c", subcore_axis_name="s", num_cores=NC, num_subcores=NS
    )

    @functools.partial(
        pl.kernel,
        out_type=jax.ShapeDtypeStruct((B * S, D), jnp.float32),
        mesh=mesh,
        scratch_types=[
            pltpu.VMEM((S, D), jnp.float32),
            pltpu.VMEM((C0,), jnp.int32),
            pltpu.VMEM((C1,), jnp.int32),
            pltpu.VMEM((C0, D), jnp.float32),
            pltpu.VMEM((C1, D), jnp.float32),
            pltpu.SemaphoreType.DMA,
        ],
        compiler_params=pltpu.CompilerParams(use_tc_tiling_on_sc=False),
    )
    def body(idx_hbm, sym_hbm, bias_hbm, out_hbm, bias_v, idx0, idx1, rows0, rows1, sem):
        wid = lax.axis_index("s") * NC + lax.axis_index("c")
        pltpu.sync_copy(bias_hbm, bias_v)
        nvec = D // LANES

        def row_body(i, carry):
            base = (wid * ROWS + i) * S
            pltpu.sync_copy(idx_hbm.at[pl.ds(base, C0)], idx0)
            cp0 = pltpu.async_copy(sym_hbm.at[idx0], rows0, sem)
            pltpu.sync_copy(idx_hbm.at[pl.ds(base + C0, C1)], idx1)
            cp1 = pltpu.async_copy(sym_hbm.at[idx1], rows1, sem)
            cp0.wait()

            def add0(r, c2):
                for c in range(nvec):
                    sl = pl.ds(c * LANES, LANES)
                    rows0[r, sl] = rows0[r, sl] + bias_v[r, sl]
                return c2

            lax.fori_loop(0, C0, add0, 0)
            pltpu.sync_copy(rows0, out_hbm.at[pl.ds(base, C0)])
            cp1.wait()

            def add1(r, c2):
                for c in range(nvec):
                    sl = pl.ds(c * LANES, LANES)
                    rows1[r, sl] = rows1[r, sl] + bias_v[C0 + r, sl]
                return c2

            lax.fori_loop(0, C1, add1, 0)
            pltpu.sync_copy(rows1, out_hbm.at[pl.ds(base + C0, C1)])
            return carry

        lax.fori_loop(0, ROWS, row_body, 0)

    return body(idx_flat, sym_table, bias)


def kernel(inputs, sym_table, pos_table):
    B, S = inputs.shape
    V, D = sym_table.shape
    bias = pl.pallas_call(
        _bias_body,
        out_shape=jax.ShapeDtypeStruct((S, D), jnp.float32),
    )(sym_table, pos_table[:S])
    idx_flat = inputs.reshape(-1).astype(jnp.int32)
    out = _sc_embed(idx_flat, sym_table, bias, B=B, S=S, D=D)
    return out.reshape(B, S, D)
